# baseline (speedup 1.0000x reference)
import functools

import jax
import jax.numpy as jnp
from jax import lax
from jax.experimental import pallas as pl
from jax.experimental.pallas import tpu as pltpu
from jax.experimental.pallas import tpu_sc as plsc

NUM_EXPERTS = 8
TOP_K = 2
LANES = 16


def _dense_kernel(x_ref, rw_ref, w1_ref, w2_ref, y_ref, gates_ref):
    e = pl.program_id(0)
    x = x_ref[...]

    @pl.when(e == 0)
    def _():
        rw = rw_ref[...]
        logits = jax.lax.dot_general(
            x, rw, (((1,), (1,)), ((), ())),
            preferred_element_type=jnp.float32)
        m = jnp.max(logits, axis=-1, keepdims=True)
        el = jnp.exp(logits - m)
        probs = el / jnp.sum(el, axis=-1, keepdims=True)
        col = jax.lax.broadcasted_iota(jnp.int32, probs.shape, 1)
        big = jnp.int32(NUM_EXPERTS)
        m1 = jnp.max(probs, axis=-1, keepdims=True)
        is1 = probs == m1
        idx1 = jnp.min(jnp.where(is1, col, big), axis=-1, keepdims=True)
        first1 = col == idx1
        probs_wo1 = jnp.where(first1, -1.0, probs)
        m2 = jnp.max(probs_wo1, axis=-1, keepdims=True)
        is2 = probs_wo1 == m2
        idx2 = jnp.min(jnp.where(is2, col, big), axis=-1, keepdims=True)
        first2 = col == idx2
        gates = jnp.where(first1 | first2, probs, 0.0)
        zpad = jnp.zeros((gates.shape[0], LANES - NUM_EXPERTS),
                         dtype=jnp.float32)
        gates_ref[...] = jnp.concatenate([gates, zpad], axis=1)

    h = jnp.dot(x, w1_ref[0], preferred_element_type=jnp.float32)
    h = jnp.maximum(h, 0.0)
    y_ref[...] = jnp.dot(h, w2_ref[0], preferred_element_type=jnp.float32)


def _dense_call(x_flat, router_w, w1, w2):
    N, C = x_flat.shape
    E, _, F = w1.shape
    return pl.pallas_call(
        _dense_kernel,
        grid=(E,),
        in_specs=[
            pl.BlockSpec((N, C), lambda e: (0, 0)),
            pl.BlockSpec((E, C), lambda e: (0, 0)),
            pl.BlockSpec((1, C, F), lambda e: (e, 0, 0)),
            pl.BlockSpec((1, F, C), lambda e: (e, 0, 0)),
        ],
        out_specs=[
            pl.BlockSpec((N, F), lambda e: (e, 0)),
            pl.BlockSpec((N, LANES), lambda e: (0, 0)),
        ],
        out_shape=[
            jax.ShapeDtypeStruct((E * N, F), jnp.float32),
            jax.ShapeDtypeStruct((N, LANES), jnp.float32),
        ],
    )(x_flat, router_w, w1, w2)


def _make_sc_combine(N, F):
    nchunks = F // LANES
    mesh = plsc.VectorSubcoreMesh(core_axis_name="c", subcore_axis_name="s")
    info = plsc.get_sparse_core_info()
    nc = info.num_cores

    @functools.partial(
        pl.kernel,
        mesh=mesh,
        out_type=jax.ShapeDtypeStruct((N, F), jnp.float32),
        scratch_types=[
            pltpu.VMEM((LANES,), jnp.float32),
            pltpu.VMEM((LANES, F), jnp.float32),
            pltpu.VMEM((F,), jnp.float32),
            pltpu.SemaphoreType.DMA,
        ],
    )
    def sc_combine(yall_hbm, gates_hbm, out_hbm, grow_v, rows_v, orow_v, sem):
        n = lax.axis_index("s") * nc + lax.axis_index("c")

        pltpu.sync_copy(gates_hbm.at[n], grow_v)

        iot = lax.iota(jnp.int32, LANES)
        idxvec = lax.rem(iot, jnp.int32(NUM_EXPERTS)) * N + n
        pltpu.async_copy(yall_hbm.at[idxvec], rows_v, sem).wait()

        gvec = grow_v[...]
        for k in range(nchunks):
            sl = pl.ds(k * LANES, LANES)
            acc = rows_v[0, sl] * gvec[0]
            for e in range(1, NUM_EXPERTS):
                acc = acc + rows_v[e, sl] * gvec[e]
            orow_v[sl] = acc
        pltpu.sync_copy(orow_v, out_hbm.at[n])

    return sc_combine


def kernel(x, router_w, w1, w2):
    B, T, C = x.shape
    N = B * T
    E, _, F = w1.shape
    x_flat = x.reshape(N, C)

    y_all, gates16 = _dense_call(x_flat, router_w, w1, w2)
    out = _make_sc_combine(N, F)(y_all, gates16)
    return out.reshape(B, T, C)


# speedup vs baseline: 1.7993x; 1.7993x over previous
import jax
import jax.numpy as jnp
from jax.experimental import pallas as pl

NUM_EXPERTS = 8
TOP_K = 2


def _moe_kernel(x_ref, rw_ref, w1_ref, w2_ref, out_ref):
    e = pl.program_id(0)
    x = x_ref[...]
    rw = rw_ref[...]

    logits = jax.lax.dot_general(
        x, rw, (((1,), (1,)), ((), ())),
        preferred_element_type=jnp.float32)
    m = jnp.max(logits, axis=-1, keepdims=True)
    el = jnp.exp(logits - m)
    probs = el / jnp.sum(el, axis=-1, keepdims=True)

    col = jax.lax.broadcasted_iota(jnp.int32, probs.shape, 1)
    big = jnp.int32(NUM_EXPERTS)
    m1 = jnp.max(probs, axis=-1, keepdims=True)
    is1 = probs == m1
    idx1 = jnp.min(jnp.where(is1, col, big), axis=-1, keepdims=True)
    first1 = col == idx1
    probs_wo1 = jnp.where(first1, -1.0, probs)
    m2 = jnp.max(probs_wo1, axis=-1, keepdims=True)
    is2 = probs_wo1 == m2
    idx2 = jnp.min(jnp.where(is2, col, big), axis=-1, keepdims=True)
    first2 = col == idx2
    gates = jnp.where(first1 | first2, probs, 0.0)

    gate_e = jnp.sum(jnp.where(col == e, gates, 0.0), axis=-1,
                     keepdims=True)

    h = jnp.dot(x, w1_ref[0], preferred_element_type=jnp.float32)
    h = jnp.maximum(h, 0.0)
    y = jnp.dot(h, w2_ref[0], preferred_element_type=jnp.float32)
    contrib = gate_e * y

    @pl.when(e == 0)
    def _():
        out_ref[...] = contrib

    @pl.when(e != 0)
    def _():
        out_ref[...] += contrib


def kernel(x, router_w, w1, w2):
    B, T, C = x.shape
    N = B * T
    E, _, F = w1.shape
    x_flat = x.reshape(N, C)

    out = pl.pallas_call(
        _moe_kernel,
        grid=(E,),
        in_specs=[
            pl.BlockSpec((N, C), lambda e: (0, 0)),
            pl.BlockSpec((E, C), lambda e: (0, 0)),
            pl.BlockSpec((1, C, F), lambda e: (e, 0, 0)),
            pl.BlockSpec((1, F, C), lambda e: (e, 0, 0)),
        ],
        out_specs=pl.BlockSpec((N, C), lambda e: (0, 0)),
        out_shape=jax.ShapeDtypeStruct((N, C), jnp.float32),
    )(x_flat, router_w, w1, w2)
    return out.reshape(B, T, C)
